# baseline (speedup 1.0000x reference)
import functools
import jax
import jax.numpy as jnp
from jax import lax
from jax.experimental import pallas as pl
from jax.experimental.pallas import tpu as pltpu
from jax.experimental.pallas import tpu_sc as plsc

N_NODES = 10000
E_EDGES = 160000
IN_DIM = 256
HID = 512
OUT = 256
K_TARGET = 1024

_I32 = jnp.int32
_F32 = jnp.float32

_HALF = 128
_CHUNK = 128
_TILES = 16
_EPAD = 163840
_CPT = _EPAD // (_TILES * _CHUNK)
_NPAD = 10240
_RPT = _NPAD // _TILES


def _sc_aggregate(yA, yB, src2d, dst2d, zrows):
    mesh = plsc.VectorSubcoreMesh(core_axis_name="c", subcore_axis_name="s")

    @functools.partial(
        pl.kernel,
        out_type=[jax.ShapeDtypeStruct((_NPAD, _HALF), _F32),
                  jax.ShapeDtypeStruct((_NPAD, _HALF), _F32)],
        mesh=mesh,
        scratch_types=[
            pltpu.VMEM((_CPT, _CHUNK), _I32),
            pltpu.VMEM((_CPT, _CHUNK), _I32),
            pltpu.VMEM((_CHUNK, _HALF), _F32),
            pltpu.VMEM_SHARED((_NPAD, _HALF), _F32),
            pltpu.SemaphoreType.DMA,
        ],
    )
    def k(yA_h, yB_h, src_h, dst_h, z_h, outA, outB,
          src_v, dst_v, rows_v, agg_sh, sem):
        c = lax.axis_index("c")
        s = lax.axis_index("s")

        def run(y_h, out_h):
            base = s * _CPT
            pltpu.sync_copy(src_h.at[pl.ds(base, _CPT)], src_v)
            pltpu.sync_copy(dst_h.at[pl.ds(base, _CPT)], dst_v)
            pltpu.sync_copy(z_h, agg_sh.at[pl.ds(s * _RPT, _RPT)])
            plsc.subcore_barrier()

            def body(j, carry):
                pltpu.async_copy(y_h.at[src_v.at[j]], rows_v, sem).wait()
                pltpu.sync_copy(rows_v, agg_sh.at[dst_v.at[j]], add=True)
                return carry

            lax.fori_loop(_I32(0), _I32(_CPT), body, 0)
            plsc.subcore_barrier()
            pltpu.sync_copy(agg_sh.at[pl.ds(s * _RPT, _RPT)],
                            out_h.at[pl.ds(s * _RPT, _RPT)])

        @pl.when(c == 0)
        def _():
            run(yA_h, outA)

        @pl.when(c == 1)
        def _():
            run(yB_h, outB)

    return k(yA, yB, src2d, dst2d, zrows)


_BIGP = 2 * E_EDGES + 1


def _sc_cluster(src_t, dst_t, keep_p, deg_p, cr_p, bgc16):
    mesh = plsc.VectorSubcoreMesh(core_axis_name="c", subcore_axis_name="s")
    EPT = E_EDGES // _TILES
    G = EPT // 16

    @functools.partial(
        pl.kernel,
        out_type=jax.ShapeDtypeStruct((_NPAD,), _I32),
        mesh=mesh,
        compiler_params=pltpu.CompilerParams(needs_layout_passes=False),
        scratch_types=[
            pltpu.VMEM((EPT,), _I32),
            pltpu.VMEM((EPT,), _I32),
            pltpu.VMEM((_NPAD,), _I32),
            pltpu.VMEM((_NPAD,), _I32),
            pltpu.VMEM((_NPAD,), _I32),
            pltpu.VMEM((_NPAD,), _I32),
            pltpu.VMEM((_NPAD,), _I32),
            pltpu.VMEM((_NPAD,), _I32),
            pltpu.VMEM((_RPT,), _I32),
            pltpu.VMEM((_RPT,), _I32),
            pltpu.VMEM((_RPT,), _I32),
            pltpu.VMEM((_RPT,), _I32),
            pltpu.VMEM((_RPT,), _I32),
            pltpu.VMEM((16,), _I32),
            pltpu.VMEM_SHARED((_TILES, _NPAD), _I32),
            pltpu.VMEM_SHARED((_TILES, _NPAD), _I32),
            pltpu.VMEM_SHARED((_NPAD,), _I32),
        ],
    )
    def k(src_h, dst_h, keep_h, deg_h, cr_h, bgc_h, out_h,
          src_v, dst_v, keep_t, deg_t, cr_t, bd, bp, bnb,
          macc, mtmp, nacc, ntmp, cid_v, bgc_v, slab, slab2, gbd, ):
        c = lax.axis_index("c")
        s = lax.axis_index("s")

        @pl.when(c == 0)
        def _():
            pltpu.sync_copy(src_h.at[s], src_v)
            pltpu.sync_copy(dst_h.at[s], dst_v)
            pltpu.sync_copy(keep_h, keep_t)
            pltpu.sync_copy(deg_h, deg_t)
            pltpu.sync_copy(cr_h, cr_t)
            pltpu.sync_copy(bgc_h, bgc_v)
            neg1 = jnp.full((16,), -1, _I32)
            bigp = jnp.full((16,), _BIGP, _I32)
            zero = jnp.zeros((16,), _I32)

            def init(i, carry):
                bd[pl.ds(i * 16, 16)] = neg1
                bp[pl.ds(i * 16, 16)] = bigp
                bnb[pl.ds(i * 16, 16)] = zero
                return carry
            lax.fori_loop(_I32(0), _I32(_NPAD // 16), init, 0)

            def smax(node, key):
                def cond(pend):
                    return jnp.max(pend) > 0

                def body(pend):
                    cur = plsc.load_gather(bd, [node])
                    need = (pend != 0) & (key > cur)
                    plsc.store_scatter(bd, [node], key, mask=need)
                    cur2 = plsc.load_gather(bd, [node])
                    return (need & (key > cur2)).astype(_I32)
                lax.while_loop(cond, body, jnp.ones((16,), _I32))

            def g1(g, carry):
                sv = src_v[pl.ds(g * 16, 16)]
                dv = dst_v[pl.ds(g * 16, 16)]
                degs = plsc.load_gather(deg_t, [sv])
                degd = plsc.load_gather(deg_t, [dv])
                kms = plsc.load_gather(keep_t, [sv])
                kmd = plsc.load_gather(keep_t, [dv])
                smax(sv, jnp.where(kmd == 1, degd, -1))
                smax(dv, jnp.where(kms == 1, degs, -1))
                return carry
            lax.fori_loop(_I32(0), _I32(G), g1, 0)

            pltpu.sync_copy(bd, slab.at[s])
            plsc.subcore_barrier()
            nb = s * _RPT
            pltpu.sync_copy(slab.at[_I32(0), pl.ds(nb, _RPT)], macc)

            def mrg1(p, carry):
                pltpu.sync_copy(slab.at[p, pl.ds(nb, _RPT)], mtmp)

                def mx(i, carry2):
                    sl = pl.ds(i * 16, 16)
                    macc[sl] = jnp.maximum(macc[sl], mtmp[sl])
                    return carry2
                lax.fori_loop(_I32(0), _I32(_RPT // 16), mx, 0)
                return carry
            lax.fori_loop(_I32(1), _I32(_TILES), mrg1, 0)
            pltpu.sync_copy(macc, gbd.at[pl.ds(nb, _RPT)])
            plsc.subcore_barrier()
            pltpu.sync_copy(gbd, bd)

            def smin(node, key, payload):
                def cond(pend):
                    return jnp.max(pend) > 0

                def body(pend):
                    cur = plsc.load_gather(bp, [node])
                    need = (pend != 0) & (key < cur)
                    plsc.store_scatter(bp, [node], key, mask=need)
                    cur2 = plsc.load_gather(bp, [node])
                    win = need & (cur2 == key)
                    plsc.store_scatter(bnb, [node], payload, mask=win)
                    return (need & (key < cur2)).astype(_I32)
                lax.while_loop(cond, body, jnp.ones((16,), _I32))

            lane = lax.iota(_I32, 16)

            def g2(g, carry):
                sv = src_v[pl.ds(g * 16, 16)]
                dv = dst_v[pl.ds(g * 16, 16)]
                degs = plsc.load_gather(deg_t, [sv])
                degd = plsc.load_gather(deg_t, [dv])
                kms = plsc.load_gather(keep_t, [sv])
                kmd = plsc.load_gather(keep_t, [dv])
                bdA = plsc.load_gather(bd, [sv])
                bdB = plsc.load_gather(bd, [dv])
                j = s * EPT + g * 16 + lane
                smin(sv, jnp.where((kmd == 1) & (degd == bdA), 2 * j, _BIGP),
                     dv)
                smin(dv, jnp.where((kms == 1) & (degs == bdB), 2 * j + 1,
                                   _BIGP), sv)
                return carry
            lax.fori_loop(_I32(0), _I32(G), g2, 0)

            pltpu.sync_copy(bp, slab.at[s])
            pltpu.sync_copy(bnb, slab2.at[s])
            plsc.subcore_barrier()
            pltpu.sync_copy(slab.at[_I32(0), pl.ds(nb, _RPT)], macc)
            pltpu.sync_copy(slab2.at[_I32(0), pl.ds(nb, _RPT)], nacc)

            def mrg2(p, carry):
                pltpu.sync_copy(slab.at[p, pl.ds(nb, _RPT)], mtmp)
                pltpu.sync_copy(slab2.at[p, pl.ds(nb, _RPT)], ntmp)

                def mn(i, carry2):
                    sl = pl.ds(i * 16, 16)
                    take = mtmp[sl] < macc[sl]
                    macc[sl] = jnp.minimum(macc[sl], mtmp[sl])
                    nacc[sl] = jnp.where(take, ntmp[sl], nacc[sl])
                    return carry2
                lax.fori_loop(_I32(0), _I32(_RPT // 16), mn, 0)
                return carry
            lax.fori_loop(_I32(1), _I32(_TILES), mrg2, 0)

            bgc = bgc_v[pl.ds(0, 16)]

            def fin(i, carry):
                sl = pl.ds(i * 16, 16)
                gsl = pl.ds(nb + i * 16, 16)
                has_c = bd[gsl] >= 0
                crn = plsc.load_gather(cr_t, [nacc[sl]])
                assigned = jnp.where(has_c, crn, bgc)
                cid_v[sl] = jnp.where(keep_t[gsl] == 1, cr_t[gsl], assigned)
                return carry
            lax.fori_loop(_I32(0), _I32(_RPT // 16), fin, 0)
            pltpu.sync_copy(cid_v, out_h.at[pl.ds(nb, _RPT)])

    return k(src_t, dst_t, keep_p, deg_p, cr_p, bgc16)


def _dense1_body(pre_ref, x_ref, dinv_ref, w1_ref, b1_ref, wsc_ref,
                 wsk_ref, bsk_ref, x1g_ref, skip_ref, raw_ref):
    dinv = dinv_ref[...]
    h = dinv * pre_ref[...] + (dinv * dinv) * x_ref[...]
    x1 = jnp.maximum(jnp.dot(h, w1_ref[...],
                             preferred_element_type=_F32) + b1_ref[...], 0.0)
    rawf = jnp.dot(x1, wsc_ref[...], preferred_element_type=_F32)
    gate = jnp.tanh(rawf[:, 0:1])
    x1g_ref[...] = x1 * gate
    skip_ref[...] = jnp.dot(x1, wsk_ref[...],
                            preferred_element_type=_F32) + bsk_ref[...]
    raw_ref[...] = rawf


def _dense1(pre, x, dinv, W1, b1, Wscore, Wskip, bskip):
    B = 1000
    grid = (N_NODES // B,)
    _i32 = lambda v: jnp.asarray(v, _I32)
    wsc_pad = jnp.zeros((HID, 128), _F32).at[:, 0:1].set(Wscore)
    out = pl.pallas_call(
        _dense1_body,
        grid=grid,
        in_specs=[
            pl.BlockSpec((B, IN_DIM), lambda i: (_i32(i), _i32(0))),
            pl.BlockSpec((B, IN_DIM), lambda i: (_i32(i), _i32(0))),
            pl.BlockSpec((B, 1), lambda i: (_i32(i), _i32(0))),
            pl.BlockSpec((IN_DIM, HID), lambda i: (_i32(0), _i32(0))),
            pl.BlockSpec((1, HID), lambda i: (_i32(0), _i32(0))),
            pl.BlockSpec((HID, 128), lambda i: (_i32(0), _i32(0))),
            pl.BlockSpec((HID, OUT), lambda i: (_i32(0), _i32(0))),
            pl.BlockSpec((1, OUT), lambda i: (_i32(0), _i32(0))),
        ],
        out_specs=[
            pl.BlockSpec((B, HID), lambda i: (_i32(i), _i32(0))),
            pl.BlockSpec((B, OUT), lambda i: (_i32(i), _i32(0))),
            pl.BlockSpec((B, 128), lambda i: (_i32(i), _i32(0))),
        ],
        out_shape=[
            jax.ShapeDtypeStruct((N_NODES, HID), _F32),
            jax.ShapeDtypeStruct((N_NODES, OUT), _F32),
            jax.ShapeDtypeStruct((N_NODES, 128), _F32),
        ],
    )(pre, x, dinv[:, None], W1, b1[None, :], wsc_pad, Wskip, bskip[None, :])
    x1g, skip, rawf = out
    return x1g, skip, rawf[:, 0]


def kernel(x, edge_index, W1, b1, W2, b2, Wskip, bskip, Wscore):
    out_dtype = jnp.result_type(x.dtype, W1.dtype)
    x = x.astype(_F32)
    W1 = W1.astype(_F32)
    b1 = b1.astype(_F32)
    W2 = W2.astype(_F32)
    b2 = b2.astype(_F32)
    Wskip = Wskip.astype(_F32)
    bskip = bskip.astype(_F32)
    Wscore = Wscore.astype(_F32)
    src = edge_index[0].astype(_I32)
    dst = edge_index[1].astype(_I32)
    N, E, K = N_NODES, E_EDGES, K_TARGET

    deg_dst = jnp.zeros((N,), _I32).at[dst].add(1)
    deg_src = jnp.zeros((N,), _I32).at[src].add(1)
    dinv = lax.rsqrt(deg_dst.astype(_F32) + 1.0)

    y = dinv[:, None] * x
    pad = _EPAD - E
    src2d = jnp.concatenate([src, jnp.zeros((pad,), _I32)]).reshape(
        _TILES * _CPT, _CHUNK)
    dst2d = jnp.concatenate([dst, jnp.full((pad,), _NPAD - 1, _I32)]).reshape(
        _TILES * _CPT, _CHUNK)
    zrows = jnp.zeros((_RPT, _HALF), _F32)
    outA, outB = _sc_aggregate(y[:, :_HALF], y[:, _HALF:], src2d, dst2d, zrows)
    agg = jnp.concatenate([outA[:N], outB[:N]], axis=1)


    x1g, skip, raw = _dense1(agg, x, dinv, W1, b1, Wscore, Wskip, bskip)

    _, kept = lax.top_k(raw, K)
    keep_mask = jnp.zeros((N,), bool).at[kept].set(True)
    cluster_rank = jnp.cumsum(keep_mask.astype(_I32)) - 1

    maxdeg = jnp.max(jnp.where(keep_mask, deg_src, -1))
    elig = keep_mask & (deg_src == maxdeg)
    bg_node = jnp.argmax(jnp.where(elig, raw, -jnp.inf))
    best_global_cluster = cluster_rank[bg_node]

    npadding = (0, _NPAD - N)
    cluster_id = _sc_cluster(
        src.reshape(_TILES, E // _TILES),
        dst.reshape(_TILES, E // _TILES),
        jnp.pad(keep_mask.astype(_I32), npadding),
        jnp.pad(deg_src, npadding),
        jnp.pad(cluster_rank, npadding),
        jnp.full((16,), best_global_cluster, _I32),
    )[:N]

    sums = jnp.zeros((K, HID), _F32).at[cluster_id].add(x1g)
    counts = jnp.zeros((K,), _I32).at[cluster_id].add(1)
    x_p = sums / jnp.maximum(counts, 1).astype(_F32)[:, None]

    cu = cluster_id[src]
    cv = cluster_id[dst]
    A = jnp.zeros((K, K), _F32).at[cu, cv].set(1.0)
    A = A * (1.0 - jnp.eye(K, dtype=_F32))
    A_hat = A + jnp.eye(K, dtype=_F32)
    degp = A_hat.sum(axis=0)
    dinvp = lax.rsqrt(degp)

    xw = x_p @ W2
    x_p2 = (A_hat * dinvp[:, None] * dinvp[None, :]).T @ xw + b2

    up = x_p2[cluster_id]
    return ((up + skip).astype(out_dtype), 0.0)


# speedup vs baseline: 2.3651x; 2.3651x over previous
import functools
import jax
import jax.numpy as jnp
from jax import lax
from jax.experimental import pallas as pl
from jax.experimental.pallas import tpu as pltpu
from jax.experimental.pallas import tpu_sc as plsc

N_NODES = 10000
E_EDGES = 160000
IN_DIM = 256
HID = 512
OUT = 256
K_TARGET = 1024

_I32 = jnp.int32
_F32 = jnp.float32

_HALF = 128
_CHUNK = 128
_TILES = 16
_EPAD = 163840
_CPT = _EPAD // (_TILES * _CHUNK)
_NPAD = 10240
_RPT = _NPAD // _TILES


def _sc_aggregate(yA, yB, src2d, dst2d, zrows):
    mesh = plsc.VectorSubcoreMesh(core_axis_name="c", subcore_axis_name="s")

    @functools.partial(
        pl.kernel,
        out_type=[jax.ShapeDtypeStruct((_NPAD, _HALF), _F32),
                  jax.ShapeDtypeStruct((_NPAD, _HALF), _F32)],
        mesh=mesh,
        scratch_types=[
            pltpu.VMEM((_CPT, _CHUNK), _I32),
            pltpu.VMEM((_CPT, _CHUNK), _I32),
            pltpu.VMEM((_CHUNK, _HALF), _F32),
            pltpu.VMEM_SHARED((_NPAD, _HALF), _F32),
            pltpu.SemaphoreType.DMA,
        ],
    )
    def k(yA_h, yB_h, src_h, dst_h, z_h, outA, outB,
          src_v, dst_v, rows_v, agg_sh, sem):
        c = lax.axis_index("c")
        s = lax.axis_index("s")

        def run(y_h, out_h):
            base = s * _CPT
            pltpu.sync_copy(src_h.at[pl.ds(base, _CPT)], src_v)
            pltpu.sync_copy(dst_h.at[pl.ds(base, _CPT)], dst_v)
            pltpu.sync_copy(z_h, agg_sh.at[pl.ds(s * _RPT, _RPT)])
            plsc.subcore_barrier()

            def body(j, carry):
                pltpu.async_copy(y_h.at[src_v.at[j]], rows_v, sem).wait()
                pltpu.sync_copy(rows_v, agg_sh.at[dst_v.at[j]], add=True)
                return carry

            lax.fori_loop(_I32(0), _I32(_CPT), body, 0)
            plsc.subcore_barrier()
            pltpu.sync_copy(agg_sh.at[pl.ds(s * _RPT, _RPT)],
                            out_h.at[pl.ds(s * _RPT, _RPT)])

        @pl.when(c == 0)
        def _():
            run(yA_h, outA)

        @pl.when(c == 1)
        def _():
            run(yB_h, outB)

    return k(yA, yB, src2d, dst2d, zrows)


_BIGP = 2 * E_EDGES + 1


def _sc_cluster(src_t, dst_t, keep_p, deg_p, cr_p, bgc16):
    mesh = plsc.VectorSubcoreMesh(core_axis_name="c", subcore_axis_name="s")
    EPT = E_EDGES // _TILES
    G = EPT // 16

    @functools.partial(
        pl.kernel,
        out_type=[jax.ShapeDtypeStruct((_NPAD,), _I32),
                  jax.ShapeDtypeStruct((_TILES, E_EDGES // _TILES), _I32),
                  jax.ShapeDtypeStruct((_TILES, E_EDGES // _TILES), _I32)],
        mesh=mesh,
        compiler_params=pltpu.CompilerParams(needs_layout_passes=False),
        scratch_types=[
            pltpu.VMEM((EPT,), _I32),
            pltpu.VMEM((EPT,), _I32),
            pltpu.VMEM((_NPAD,), _I32),
            pltpu.VMEM((_NPAD,), _I32),
            pltpu.VMEM((_NPAD,), _I32),
            pltpu.VMEM((_NPAD,), _I32),
            pltpu.VMEM((_NPAD,), _I32),
            pltpu.VMEM((_NPAD,), _I32),
            pltpu.VMEM((_RPT,), _I32),
            pltpu.VMEM((_RPT,), _I32),
            pltpu.VMEM((_RPT,), _I32),
            pltpu.VMEM((_RPT,), _I32),
            pltpu.VMEM((_RPT,), _I32),
            pltpu.VMEM((16,), _I32),
            pltpu.VMEM((E_EDGES // _TILES,), _I32),
            pltpu.VMEM((E_EDGES // _TILES,), _I32),
            pltpu.VMEM_SHARED((_TILES, _NPAD), _I32),
            pltpu.VMEM_SHARED((_TILES, _NPAD), _I32),
            pltpu.VMEM_SHARED((_NPAD,), _I32),
        ],
    )
    def k(src_h, dst_h, keep_h, deg_h, cr_h, bgc_h, out_h, cu_h, cv_h,
          src_v, dst_v, keep_t, deg_t, cr_t, bd, bp, bnb,
          macc, mtmp, nacc, ntmp, cid_v, bgc_v, cu_v, cv_v,
          slab, slab2, gbd, ):
        c = lax.axis_index("c")
        s = lax.axis_index("s")

        @pl.when(c == 0)
        def _():
            pltpu.sync_copy(src_h.at[s], src_v)
            pltpu.sync_copy(dst_h.at[s], dst_v)
            pltpu.sync_copy(keep_h, keep_t)
            pltpu.sync_copy(deg_h, deg_t)
            pltpu.sync_copy(cr_h, cr_t)
            pltpu.sync_copy(bgc_h, bgc_v)
            neg1 = jnp.full((16,), -1, _I32)
            bigp = jnp.full((16,), _BIGP, _I32)
            zero = jnp.zeros((16,), _I32)

            def init(i, carry):
                bd[pl.ds(i * 16, 16)] = neg1
                bp[pl.ds(i * 16, 16)] = bigp
                bnb[pl.ds(i * 16, 16)] = zero
                return carry
            lax.fori_loop(_I32(0), _I32(_NPAD // 16), init, 0)

            def smax(node, key):
                def cond(pend):
                    return jnp.max(pend) > 0

                def body(pend):
                    cur = plsc.load_gather(bd, [node])
                    need = (pend != 0) & (key > cur)
                    plsc.store_scatter(bd, [node], key, mask=need)
                    cur2 = plsc.load_gather(bd, [node])
                    return (need & (key > cur2)).astype(_I32)
                lax.while_loop(cond, body, jnp.ones((16,), _I32))

            def g1(g, carry):
                sv = src_v[pl.ds(g * 16, 16)]
                dv = dst_v[pl.ds(g * 16, 16)]
                degs = plsc.load_gather(deg_t, [sv])
                degd = plsc.load_gather(deg_t, [dv])
                kms = plsc.load_gather(keep_t, [sv])
                kmd = plsc.load_gather(keep_t, [dv])
                smax(sv, jnp.where(kmd == 1, degd, -1))
                smax(dv, jnp.where(kms == 1, degs, -1))
                return carry
            lax.fori_loop(_I32(0), _I32(G), g1, 0)

            pltpu.sync_copy(bd, slab.at[s])
            plsc.subcore_barrier()
            nb = s * _RPT
            pltpu.sync_copy(slab.at[_I32(0), pl.ds(nb, _RPT)], macc)

            def mrg1(p, carry):
                pltpu.sync_copy(slab.at[p, pl.ds(nb, _RPT)], mtmp)

                def mx(i, carry2):
                    sl = pl.ds(i * 16, 16)
                    macc[sl] = jnp.maximum(macc[sl], mtmp[sl])
                    return carry2
                lax.fori_loop(_I32(0), _I32(_RPT // 16), mx, 0)
                return carry
            lax.fori_loop(_I32(1), _I32(_TILES), mrg1, 0)
            pltpu.sync_copy(macc, gbd.at[pl.ds(nb, _RPT)])
            plsc.subcore_barrier()
            pltpu.sync_copy(gbd, bd)

            def smin(node, key, payload):
                def cond(pend):
                    return jnp.max(pend) > 0

                def body(pend):
                    cur = plsc.load_gather(bp, [node])
                    need = (pend != 0) & (key < cur)
                    plsc.store_scatter(bp, [node], key, mask=need)
                    cur2 = plsc.load_gather(bp, [node])
                    win = need & (cur2 == key)
                    plsc.store_scatter(bnb, [node], payload, mask=win)
                    return (need & (key < cur2)).astype(_I32)
                lax.while_loop(cond, body, jnp.ones((16,), _I32))

            lane = lax.iota(_I32, 16)

            def g2(g, carry):
                sv = src_v[pl.ds(g * 16, 16)]
                dv = dst_v[pl.ds(g * 16, 16)]
                degs = plsc.load_gather(deg_t, [sv])
                degd = plsc.load_gather(deg_t, [dv])
                kms = plsc.load_gather(keep_t, [sv])
                kmd = plsc.load_gather(keep_t, [dv])
                bdA = plsc.load_gather(bd, [sv])
                bdB = plsc.load_gather(bd, [dv])
                j = s * EPT + g * 16 + lane
                smin(sv, jnp.where((kmd == 1) & (degd == bdA), 2 * j, _BIGP),
                     dv)
                smin(dv, jnp.where((kms == 1) & (degs == bdB), 2 * j + 1,
                                   _BIGP), sv)
                return carry
            lax.fori_loop(_I32(0), _I32(G), g2, 0)

            pltpu.sync_copy(bp, slab.at[s])
            pltpu.sync_copy(bnb, slab2.at[s])
            plsc.subcore_barrier()
            pltpu.sync_copy(slab.at[_I32(0), pl.ds(nb, _RPT)], macc)
            pltpu.sync_copy(slab2.at[_I32(0), pl.ds(nb, _RPT)], nacc)

            def mrg2(p, carry):
                pltpu.sync_copy(slab.at[p, pl.ds(nb, _RPT)], mtmp)
                pltpu.sync_copy(slab2.at[p, pl.ds(nb, _RPT)], ntmp)

                def mn(i, carry2):
                    sl = pl.ds(i * 16, 16)
                    take = mtmp[sl] < macc[sl]
                    macc[sl] = jnp.minimum(macc[sl], mtmp[sl])
                    nacc[sl] = jnp.where(take, ntmp[sl], nacc[sl])
                    return carry2
                lax.fori_loop(_I32(0), _I32(_RPT // 16), mn, 0)
                return carry
            lax.fori_loop(_I32(1), _I32(_TILES), mrg2, 0)

            bgc = bgc_v[pl.ds(0, 16)]

            def fin(i, carry):
                sl = pl.ds(i * 16, 16)
                gsl = pl.ds(nb + i * 16, 16)
                has_c = bd[gsl] >= 0
                crn = plsc.load_gather(cr_t, [nacc[sl]])
                assigned = jnp.where(has_c, crn, bgc)
                cid_v[sl] = jnp.where(keep_t[gsl] == 1, cr_t[gsl], assigned)
                return carry
            lax.fori_loop(_I32(0), _I32(_RPT // 16), fin, 0)
            pltpu.sync_copy(cid_v, out_h.at[pl.ds(nb, _RPT)])

            pltpu.sync_copy(cid_v, gbd.at[pl.ds(nb, _RPT)])
            plsc.subcore_barrier()
            pltpu.sync_copy(gbd, bd)

            def ge(g, carry):
                sl = pl.ds(g * 16, 16)
                cu_v[sl] = plsc.load_gather(bd, [src_v[sl]])
                cv_v[sl] = plsc.load_gather(bd, [dst_v[sl]])
                return carry
            lax.fori_loop(_I32(0), _I32(G), ge, 0)
            pltpu.sync_copy(cu_v, cu_h.at[s])
            pltpu.sync_copy(cv_v, cv_h.at[s])

    return k(src_t, dst_t, keep_p, deg_p, cr_p, bgc16)


def _dense1_body(pre_ref, x_ref, dinv_ref, w1_ref, b1_ref, wsc_ref,
                 wsk_ref, bsk_ref, x1g_ref, skip_ref, raw_ref):
    dinv = dinv_ref[...]
    h = dinv * pre_ref[...] + (dinv * dinv) * x_ref[...]
    x1 = jnp.maximum(jnp.dot(h, w1_ref[...],
                             preferred_element_type=_F32) + b1_ref[...], 0.0)
    rawf = jnp.dot(x1, wsc_ref[...], preferred_element_type=_F32)
    gate = jnp.tanh(rawf[:, 0:1])
    x1g_ref[...] = x1 * gate
    skip_ref[...] = jnp.dot(x1, wsk_ref[...],
                            preferred_element_type=_F32) + bsk_ref[...]
    raw_ref[...] = rawf


def _dense1(pre, x, dinv, W1, b1, Wscore, Wskip, bskip):
    B = 1000
    grid = (N_NODES // B,)
    _i32 = lambda v: jnp.asarray(v, _I32)
    wsc_pad = jnp.zeros((HID, 128), _F32).at[:, 0:1].set(Wscore)
    out = pl.pallas_call(
        _dense1_body,
        grid=grid,
        in_specs=[
            pl.BlockSpec((B, IN_DIM), lambda i: (_i32(i), _i32(0))),
            pl.BlockSpec((B, IN_DIM), lambda i: (_i32(i), _i32(0))),
            pl.BlockSpec((B, 1), lambda i: (_i32(i), _i32(0))),
            pl.BlockSpec((IN_DIM, HID), lambda i: (_i32(0), _i32(0))),
            pl.BlockSpec((1, HID), lambda i: (_i32(0), _i32(0))),
            pl.BlockSpec((HID, 128), lambda i: (_i32(0), _i32(0))),
            pl.BlockSpec((HID, OUT), lambda i: (_i32(0), _i32(0))),
            pl.BlockSpec((1, OUT), lambda i: (_i32(0), _i32(0))),
        ],
        out_specs=[
            pl.BlockSpec((B, HID), lambda i: (_i32(i), _i32(0))),
            pl.BlockSpec((B, OUT), lambda i: (_i32(i), _i32(0))),
            pl.BlockSpec((B, 128), lambda i: (_i32(i), _i32(0))),
        ],
        out_shape=[
            jax.ShapeDtypeStruct((N_NODES, HID), _F32),
            jax.ShapeDtypeStruct((N_NODES, OUT), _F32),
            jax.ShapeDtypeStruct((N_NODES, 128), _F32),
        ],
    )(pre, x, dinv[:, None], W1, b1[None, :], wsc_pad, Wskip, bskip[None, :])
    x1g, skip, rawf = out
    return x1g, skip, rawf[:, 0]


def _pool_body(cid_ref, x1g_ref, sums_ref, cnt_ref):
    i = pl.program_id(0)

    @pl.when(i == 0)
    def _():
        sums_ref[...] = jnp.zeros_like(sums_ref)
        cnt_ref[...] = jnp.zeros_like(cnt_ref)

    onehot = (lax.broadcasted_iota(_I32, (cid_ref.shape[0], K_TARGET), 1)
              == cid_ref[...]).astype(_F32)
    sums_ref[...] += lax.dot_general(onehot, x1g_ref[...],
                                     (((0,), (0,)), ((), ())),
                                     preferred_element_type=_F32)
    cnt_ref[...] += jnp.sum(onehot, axis=0)[None, :]


def _pool(cluster_id, x1g):
    B = 1000
    _i32 = lambda v: jnp.asarray(v, _I32)
    return pl.pallas_call(
        _pool_body,
        grid=(N_NODES // B,),
        in_specs=[
            pl.BlockSpec((B, 1), lambda i: (_i32(i), _i32(0))),
            pl.BlockSpec((B, HID), lambda i: (_i32(i), _i32(0))),
        ],
        out_specs=[
            pl.BlockSpec((K_TARGET, HID), lambda i: (_i32(0), _i32(0))),
            pl.BlockSpec((1, K_TARGET), lambda i: (_i32(0), _i32(0))),
        ],
        out_shape=[
            jax.ShapeDtypeStruct((K_TARGET, HID), _F32),
            jax.ShapeDtypeStruct((1, K_TARGET), _F32),
        ],
        compiler_params=pltpu.CompilerParams(
            dimension_semantics=("arbitrary",)),
    )(cluster_id[:, None], x1g)


def _pgcn_body(a_ref, sums_ref, cnt_ref, w2_ref, b2_ref, out_ref):
    K = K_TARGET
    ii = lax.broadcasted_iota(_I32, (K, K), 0)
    jj = lax.broadcasted_iota(_I32, (K, K), 1)
    a_hat = jnp.where(ii == jj, 1.0, a_ref[...])
    degp = jnp.sum(a_hat, axis=0)
    dinvp = lax.rsqrt(degp)
    recip = 1.0 / jnp.maximum(cnt_ref[0, :], 1.0)
    x_p = sums_ref[...] * recip[:, None]
    xw = jnp.dot(x_p, w2_ref[...], preferred_element_type=_F32)
    z = dinvp[:, None] * xw
    x_p2 = dinvp[:, None] * lax.dot_general(
        a_hat, z, (((0,), (0,)), ((), ())), preferred_element_type=_F32)
    out_ref[...] = x_p2 + b2_ref[...]


def _pgcn(A, sums, cnt, W2, b2):
    K = K_TARGET
    return pl.pallas_call(
        _pgcn_body,
        out_shape=jax.ShapeDtypeStruct((K, OUT), _F32),
    )(A, sums, cnt, W2, b2[None, :])


def _bcast_body(cid_ref, xp2_ref, skip_ref, out_ref):
    onehot = (lax.broadcasted_iota(_I32, (cid_ref.shape[0], K_TARGET), 1)
              == cid_ref[...]).astype(_F32)
    out_ref[...] = jnp.dot(onehot, xp2_ref[...],
                           preferred_element_type=_F32) + skip_ref[...]


def _bcast(cluster_id, x_p2, skip):
    B = 1000
    _i32 = lambda v: jnp.asarray(v, _I32)
    return pl.pallas_call(
        _bcast_body,
        grid=(N_NODES // B,),
        in_specs=[
            pl.BlockSpec((B, 1), lambda i: (_i32(i), _i32(0))),
            pl.BlockSpec((K_TARGET, OUT), lambda i: (_i32(0), _i32(0))),
            pl.BlockSpec((B, OUT), lambda i: (_i32(i), _i32(0))),
        ],
        out_specs=pl.BlockSpec((B, OUT), lambda i: (_i32(i), _i32(0))),
        out_shape=jax.ShapeDtypeStruct((N_NODES, OUT), _F32),
    )(cluster_id[:, None], x_p2, skip)


def kernel(x, edge_index, W1, b1, W2, b2, Wskip, bskip, Wscore):
    out_dtype = jnp.result_type(x.dtype, W1.dtype)
    x = x.astype(_F32)
    W1 = W1.astype(_F32)
    b1 = b1.astype(_F32)
    W2 = W2.astype(_F32)
    b2 = b2.astype(_F32)
    Wskip = Wskip.astype(_F32)
    bskip = bskip.astype(_F32)
    Wscore = Wscore.astype(_F32)
    src = edge_index[0].astype(_I32)
    dst = edge_index[1].astype(_I32)
    N, E, K = N_NODES, E_EDGES, K_TARGET

    deg_dst = jnp.zeros((N,), _I32).at[dst].add(1)
    deg_src = jnp.zeros((N,), _I32).at[src].add(1)
    dinv = lax.rsqrt(deg_dst.astype(_F32) + 1.0)

    y = dinv[:, None] * x
    pad = _EPAD - E
    src2d = jnp.concatenate([src, jnp.zeros((pad,), _I32)]).reshape(
        _TILES * _CPT, _CHUNK)
    dst2d = jnp.concatenate([dst, jnp.full((pad,), _NPAD - 1, _I32)]).reshape(
        _TILES * _CPT, _CHUNK)
    zrows = jnp.zeros((_RPT, _HALF), _F32)
    outA, outB = _sc_aggregate(y[:, :_HALF], y[:, _HALF:], src2d, dst2d, zrows)
    agg = jnp.concatenate([outA[:N], outB[:N]], axis=1)


    x1g, skip, raw = _dense1(agg, x, dinv, W1, b1, Wscore, Wskip, bskip)

    _, kept = lax.top_k(raw, K)
    keep_mask = jnp.zeros((N,), bool).at[kept].set(True)
    cluster_rank = jnp.cumsum(keep_mask.astype(_I32)) - 1

    maxdeg = jnp.max(jnp.where(keep_mask, deg_src, -1))
    elig = keep_mask & (deg_src == maxdeg)
    bg_node = jnp.argmax(jnp.where(elig, raw, -jnp.inf))
    best_global_cluster = cluster_rank[bg_node]

    npadding = (0, _NPAD - N)
    cid_pad, cu2d, cv2d = _sc_cluster(
        src.reshape(_TILES, E // _TILES),
        dst.reshape(_TILES, E // _TILES),
        jnp.pad(keep_mask.astype(_I32), npadding),
        jnp.pad(deg_src, npadding),
        jnp.pad(cluster_rank, npadding),
        jnp.full((16,), best_global_cluster, _I32),
    )
    cluster_id = cid_pad[:N]

    sums, cnt = _pool(cluster_id, x1g)

    A = jnp.zeros((K, K), _F32).at[cu2d.reshape(-1), cv2d.reshape(-1)].set(1.0)
    x_p2 = _pgcn(A, sums, cnt, W2, b2)

    out = _bcast(cluster_id, x_p2, skip)
    return (out.astype(out_dtype), 0.0)


# speedup vs baseline: 3.4571x; 1.4617x over previous
import functools
import jax
import jax.numpy as jnp
from jax import lax
from jax.experimental import pallas as pl
from jax.experimental.pallas import tpu as pltpu
from jax.experimental.pallas import tpu_sc as plsc

N_NODES = 10000
E_EDGES = 160000
IN_DIM = 256
HID = 512
OUT = 256
K_TARGET = 1024

_I32 = jnp.int32
_F32 = jnp.float32

_HALF = 128
_CHUNK = 128
_TILES = 16
_EPAD = 163840
_CPT = _EPAD // (_TILES * _CHUNK)
_NPAD = 10240
_RPT = _NPAD // _TILES


def _sc_aggregate(yA, yB, src2d, dst2d, zrows):
    mesh = plsc.VectorSubcoreMesh(core_axis_name="c", subcore_axis_name="s")

    @functools.partial(
        pl.kernel,
        out_type=[jax.ShapeDtypeStruct((_NPAD, _HALF), _F32),
                  jax.ShapeDtypeStruct((_NPAD, _HALF), _F32)],
        mesh=mesh,
        scratch_types=[
            pltpu.VMEM((_CPT, _CHUNK), _I32),
            pltpu.VMEM((_CPT, _CHUNK), _I32),
            pltpu.VMEM((_CHUNK, _HALF), _F32),
            pltpu.VMEM_SHARED((_NPAD, _HALF), _F32),
            pltpu.SemaphoreType.DMA,
        ],
    )
    def k(yA_h, yB_h, src_h, dst_h, z_h, outA, outB,
          src_v, dst_v, rows_v, agg_sh, sem):
        c = lax.axis_index("c")
        s = lax.axis_index("s")

        def run(y_h, out_h):
            base = s * _CPT
            pltpu.sync_copy(src_h.at[pl.ds(base, _CPT)], src_v)
            pltpu.sync_copy(dst_h.at[pl.ds(base, _CPT)], dst_v)
            pltpu.sync_copy(z_h, agg_sh.at[pl.ds(s * _RPT, _RPT)])
            plsc.subcore_barrier()

            def body(j, carry):
                pltpu.async_copy(y_h.at[src_v.at[j]], rows_v, sem).wait()
                pltpu.sync_copy(rows_v, agg_sh.at[dst_v.at[j]], add=True)
                return carry

            lax.fori_loop(_I32(0), _I32(_CPT), body, 0)
            plsc.subcore_barrier()
            pltpu.sync_copy(agg_sh.at[pl.ds(s * _RPT, _RPT)],
                            out_h.at[pl.ds(s * _RPT, _RPT)])

        @pl.when(c == 0)
        def _():
            run(yA_h, outA)

        @pl.when(c == 1)
        def _():
            run(yB_h, outB)

    return k(yA, yB, src2d, dst2d, zrows)


_BIGP = 2 * E_EDGES + 1


def _sc_cluster(src_t, dst_t, keep_p, deg_p, cr_p, bgc16):
    mesh = plsc.VectorSubcoreMesh(core_axis_name="c", subcore_axis_name="s")
    EPT = E_EDGES // _TILES
    G = EPT // 16

    @functools.partial(
        pl.kernel,
        out_type=[jax.ShapeDtypeStruct((_NPAD,), _I32),
                  jax.ShapeDtypeStruct((_TILES, E_EDGES // _TILES), _I32),
                  jax.ShapeDtypeStruct((_TILES, E_EDGES // _TILES), _I32)],
        mesh=mesh,
        compiler_params=pltpu.CompilerParams(needs_layout_passes=False),
        scratch_types=[
            pltpu.VMEM((EPT,), _I32),
            pltpu.VMEM((EPT,), _I32),
            pltpu.VMEM((_NPAD,), _I32),
            pltpu.VMEM((_NPAD,), _I32),
            pltpu.VMEM((_NPAD,), _I32),
            pltpu.VMEM((_NPAD,), _I32),
            pltpu.VMEM((_NPAD,), _I32),
            pltpu.VMEM((_NPAD,), _I32),
            pltpu.VMEM((_RPT,), _I32),
            pltpu.VMEM((_RPT,), _I32),
            pltpu.VMEM((_RPT,), _I32),
            pltpu.VMEM((_RPT,), _I32),
            pltpu.VMEM((_RPT,), _I32),
            pltpu.VMEM((16,), _I32),
            pltpu.VMEM((E_EDGES // _TILES,), _I32),
            pltpu.VMEM((E_EDGES // _TILES,), _I32),
            pltpu.VMEM_SHARED((_TILES, _NPAD), _I32),
            pltpu.VMEM_SHARED((_TILES, _NPAD), _I32),
            pltpu.VMEM_SHARED((_NPAD,), _I32),
        ],
    )
    def k(src_h, dst_h, keep_h, deg_h, cr_h, bgc_h, out_h, cu_h, cv_h,
          src_v, dst_v, keep_t, deg_t, cr_t, bd, bp, bnb,
          macc, mtmp, nacc, ntmp, cid_v, bgc_v, cu_v, cv_v,
          slab, slab2, gbd, ):
        c = lax.axis_index("c")
        s = lax.axis_index("s")

        @pl.when(c == 0)
        def _():
            pltpu.sync_copy(src_h.at[s], src_v)
            pltpu.sync_copy(dst_h.at[s], dst_v)
            pltpu.sync_copy(keep_h, keep_t)
            pltpu.sync_copy(deg_h, deg_t)
            pltpu.sync_copy(cr_h, cr_t)
            pltpu.sync_copy(bgc_h, bgc_v)
            neg1 = jnp.full((16,), -1, _I32)
            bigp = jnp.full((16,), _BIGP, _I32)
            zero = jnp.zeros((16,), _I32)

            def init(i, carry):
                bd[pl.ds(i * 16, 16)] = neg1
                bp[pl.ds(i * 16, 16)] = bigp
                bnb[pl.ds(i * 16, 16)] = zero
                return carry
            lax.fori_loop(_I32(0), _I32(_NPAD // 16), init, 0)

            def smax(node, key):
                def cond(pend):
                    return jnp.max(pend) > 0

                def body(pend):
                    cur = plsc.load_gather(bd, [node])
                    need = (pend != 0) & (key > cur)
                    plsc.store_scatter(bd, [node], key, mask=need)
                    cur2 = plsc.load_gather(bd, [node])
                    return (need & (key > cur2)).astype(_I32)
                lax.while_loop(cond, body, jnp.ones((16,), _I32))

            def g1(g, carry):
                sv = src_v[pl.ds(g * 16, 16)]
                dv = dst_v[pl.ds(g * 16, 16)]
                degs = plsc.load_gather(deg_t, [sv])
                degd = plsc.load_gather(deg_t, [dv])
                kms = plsc.load_gather(keep_t, [sv])
                kmd = plsc.load_gather(keep_t, [dv])
                smax(sv, jnp.where(kmd == 1, degd, -1))
                smax(dv, jnp.where(kms == 1, degs, -1))
                return carry
            lax.fori_loop(_I32(0), _I32(G), g1, 0)

            pltpu.sync_copy(bd, slab.at[s])
            plsc.subcore_barrier()
            nb = s * _RPT
            pltpu.sync_copy(slab.at[_I32(0), pl.ds(nb, _RPT)], macc)

            def mrg1(p, carry):
                pltpu.sync_copy(slab.at[p, pl.ds(nb, _RPT)], mtmp)

                def mx(i, carry2):
                    sl = pl.ds(i * 16, 16)
                    macc[sl] = jnp.maximum(macc[sl], mtmp[sl])
                    return carry2
                lax.fori_loop(_I32(0), _I32(_RPT // 16), mx, 0)
                return carry
            lax.fori_loop(_I32(1), _I32(_TILES), mrg1, 0)
            pltpu.sync_copy(macc, gbd.at[pl.ds(nb, _RPT)])
            plsc.subcore_barrier()
            pltpu.sync_copy(gbd, bd)

            def smin(node, key, payload):
                def cond(pend):
                    return jnp.max(pend) > 0

                def body(pend):
                    cur = plsc.load_gather(bp, [node])
                    need = (pend != 0) & (key < cur)
                    plsc.store_scatter(bp, [node], key, mask=need)
                    cur2 = plsc.load_gather(bp, [node])
                    win = need & (cur2 == key)
                    plsc.store_scatter(bnb, [node], payload, mask=win)
                    return (need & (key < cur2)).astype(_I32)
                lax.while_loop(cond, body, jnp.ones((16,), _I32))

            lane = lax.iota(_I32, 16)

            def g2(g, carry):
                sv = src_v[pl.ds(g * 16, 16)]
                dv = dst_v[pl.ds(g * 16, 16)]
                degs = plsc.load_gather(deg_t, [sv])
                degd = plsc.load_gather(deg_t, [dv])
                kms = plsc.load_gather(keep_t, [sv])
                kmd = plsc.load_gather(keep_t, [dv])
                bdA = plsc.load_gather(bd, [sv])
                bdB = plsc.load_gather(bd, [dv])
                j = s * EPT + g * 16 + lane
                smin(sv, jnp.where((kmd == 1) & (degd == bdA), 2 * j, _BIGP),
                     dv)
                smin(dv, jnp.where((kms == 1) & (degs == bdB), 2 * j + 1,
                                   _BIGP), sv)
                return carry
            lax.fori_loop(_I32(0), _I32(G), g2, 0)

            pltpu.sync_copy(bp, slab.at[s])
            pltpu.sync_copy(bnb, slab2.at[s])
            plsc.subcore_barrier()
            pltpu.sync_copy(slab.at[_I32(0), pl.ds(nb, _RPT)], macc)
            pltpu.sync_copy(slab2.at[_I32(0), pl.ds(nb, _RPT)], nacc)

            def mrg2(p, carry):
                pltpu.sync_copy(slab.at[p, pl.ds(nb, _RPT)], mtmp)
                pltpu.sync_copy(slab2.at[p, pl.ds(nb, _RPT)], ntmp)

                def mn(i, carry2):
                    sl = pl.ds(i * 16, 16)
                    take = mtmp[sl] < macc[sl]
                    macc[sl] = jnp.minimum(macc[sl], mtmp[sl])
                    nacc[sl] = jnp.where(take, ntmp[sl], nacc[sl])
                    return carry2
                lax.fori_loop(_I32(0), _I32(_RPT // 16), mn, 0)
                return carry
            lax.fori_loop(_I32(1), _I32(_TILES), mrg2, 0)

            bgc = bgc_v[pl.ds(0, 16)]

            def fin(i, carry):
                sl = pl.ds(i * 16, 16)
                gsl = pl.ds(nb + i * 16, 16)
                has_c = bd[gsl] >= 0
                crn = plsc.load_gather(cr_t, [nacc[sl]])
                assigned = jnp.where(has_c, crn, bgc)
                cid_v[sl] = jnp.where(keep_t[gsl] == 1, cr_t[gsl], assigned)
                return carry
            lax.fori_loop(_I32(0), _I32(_RPT // 16), fin, 0)
            pltpu.sync_copy(cid_v, out_h.at[pl.ds(nb, _RPT)])

            pltpu.sync_copy(cid_v, gbd.at[pl.ds(nb, _RPT)])
            plsc.subcore_barrier()
            pltpu.sync_copy(gbd, bd)

            def ge(g, carry):
                sl = pl.ds(g * 16, 16)
                cu_v[sl] = plsc.load_gather(bd, [src_v[sl]])
                cv_v[sl] = plsc.load_gather(bd, [dst_v[sl]])
                return carry
            lax.fori_loop(_I32(0), _I32(G), ge, 0)
            pltpu.sync_copy(cu_v, cu_h.at[s])
            pltpu.sync_copy(cv_v, cv_h.at[s])

    return k(src_t, dst_t, keep_p, deg_p, cr_p, bgc16)


def _dense1_body(pre_ref, x_ref, dinv_ref, w1_ref, b1_ref, wsc_ref,
                 wsk_ref, bsk_ref, x1g_ref, skip_ref, raw_ref):
    dinv = dinv_ref[...]
    h = dinv * pre_ref[...] + (dinv * dinv) * x_ref[...]
    x1 = jnp.maximum(jnp.dot(h, w1_ref[...],
                             preferred_element_type=_F32) + b1_ref[...], 0.0)
    rawf = jnp.dot(x1, wsc_ref[...], preferred_element_type=_F32)
    gate = jnp.tanh(rawf[:, 0:1])
    x1g_ref[...] = x1 * gate
    skip_ref[...] = jnp.dot(x1, wsk_ref[...],
                            preferred_element_type=_F32) + bsk_ref[...]
    raw_ref[...] = rawf


def _dense1(pre, x, dinv, W1, b1, Wscore, Wskip, bskip):
    B = 1000
    grid = (N_NODES // B,)
    _i32 = lambda v: jnp.asarray(v, _I32)
    wsc_pad = jnp.zeros((HID, 128), _F32).at[:, 0:1].set(Wscore)
    out = pl.pallas_call(
        _dense1_body,
        grid=grid,
        in_specs=[
            pl.BlockSpec((B, IN_DIM), lambda i: (_i32(i), _i32(0))),
            pl.BlockSpec((B, IN_DIM), lambda i: (_i32(i), _i32(0))),
            pl.BlockSpec((B, 1), lambda i: (_i32(i), _i32(0))),
            pl.BlockSpec((IN_DIM, HID), lambda i: (_i32(0), _i32(0))),
            pl.BlockSpec((1, HID), lambda i: (_i32(0), _i32(0))),
            pl.BlockSpec((HID, 128), lambda i: (_i32(0), _i32(0))),
            pl.BlockSpec((HID, OUT), lambda i: (_i32(0), _i32(0))),
            pl.BlockSpec((1, OUT), lambda i: (_i32(0), _i32(0))),
        ],
        out_specs=[
            pl.BlockSpec((B, HID), lambda i: (_i32(i), _i32(0))),
            pl.BlockSpec((B, OUT), lambda i: (_i32(i), _i32(0))),
            pl.BlockSpec((B, 128), lambda i: (_i32(i), _i32(0))),
        ],
        out_shape=[
            jax.ShapeDtypeStruct((N_NODES, HID), _F32),
            jax.ShapeDtypeStruct((N_NODES, OUT), _F32),
            jax.ShapeDtypeStruct((N_NODES, 128), _F32),
        ],
    )(pre, x, dinv[:, None], W1, b1[None, :], wsc_pad, Wskip, bskip[None, :])
    x1g, skip, rawf = out
    return x1g, skip, rawf[:, 0]


def _sc_adj(cu2d, cv2d):
    mesh = plsc.VectorSubcoreMesh(core_axis_name="c", subcore_axis_name="s")
    EPT = E_EDGES // _TILES
    G = EPT // 16
    ROWS = K_TARGET // 32

    @functools.partial(
        pl.kernel,
        out_type=jax.ShapeDtypeStruct((K_TARGET, K_TARGET), _F32),
        mesh=mesh,
        compiler_params=pltpu.CompilerParams(needs_layout_passes=False),
        scratch_types=[
            pltpu.VMEM((EPT,), _I32),
            pltpu.VMEM((EPT,), _I32),
            pltpu.VMEM((ROWS, K_TARGET), _F32),
        ],
    )
    def k(cu_h, cv_h, a_h, cu_v, cv_v, a_loc):
        c = lax.axis_index("c")
        s = lax.axis_index("s")
        w = s * 2 + c
        lo = w * ROWS
        zero16 = jnp.zeros((16,), _F32)
        one16 = jnp.ones((16,), _F32)

        def z2(i, carry):
            r = i // (K_TARGET // 16)
            col = (i % (K_TARGET // 16)) * 16
            a_loc[r, pl.ds(col, 16)] = zero16
            return carry
        lax.fori_loop(_I32(0), _I32(ROWS * (K_TARGET // 16)), z2, 0)

        def tloop(t, carry):
            pltpu.sync_copy(cu_h.at[t], cu_v)
            pltpu.sync_copy(cv_h.at[t], cv_v)

            def g(gi, carry2):
                sl = pl.ds(gi * 16, 16)
                cu = cu_v[sl]
                cv = cv_v[sl]
                m = (cu >= lo) & (cu < lo + ROWS)
                cur = jnp.where(m, cu - lo, 0)
                plsc.store_scatter(a_loc, [cur, cv], one16, mask=m)
                return carry2
            lax.fori_loop(_I32(0), _I32(G), g, 0)
            return carry
        lax.fori_loop(_I32(0), _I32(_TILES), tloop, 0)
        pltpu.sync_copy(a_loc, a_h.at[pl.ds(lo, ROWS)])

    return k(cu2d, cv2d)


def _pool_body(cid_ref, x1g_ref, sums_ref, cnt_ref):
    i = pl.program_id(0)

    @pl.when(i == 0)
    def _():
        sums_ref[...] = jnp.zeros_like(sums_ref)
        cnt_ref[...] = jnp.zeros_like(cnt_ref)

    onehot = (lax.broadcasted_iota(_I32, (cid_ref.shape[0], K_TARGET), 1)
              == cid_ref[...]).astype(_F32)
    sums_ref[...] += lax.dot_general(onehot, x1g_ref[...],
                                     (((0,), (0,)), ((), ())),
                                     preferred_element_type=_F32)
    cnt_ref[...] += jnp.sum(onehot, axis=0)[None, :]


def _pool(cluster_id, x1g):
    B = 1000
    _i32 = lambda v: jnp.asarray(v, _I32)
    return pl.pallas_call(
        _pool_body,
        grid=(N_NODES // B,),
        in_specs=[
            pl.BlockSpec((B, 1), lambda i: (_i32(i), _i32(0))),
            pl.BlockSpec((B, HID), lambda i: (_i32(i), _i32(0))),
        ],
        out_specs=[
            pl.BlockSpec((K_TARGET, HID), lambda i: (_i32(0), _i32(0))),
            pl.BlockSpec((1, K_TARGET), lambda i: (_i32(0), _i32(0))),
        ],
        out_shape=[
            jax.ShapeDtypeStruct((K_TARGET, HID), _F32),
            jax.ShapeDtypeStruct((1, K_TARGET), _F32),
        ],
        compiler_params=pltpu.CompilerParams(
            dimension_semantics=("arbitrary",)),
    )(cluster_id[:, None], x1g)


def _pgcn_body(a_ref, sums_ref, cnt_ref, w2_ref, b2_ref, out_ref):
    K = K_TARGET
    ii = lax.broadcasted_iota(_I32, (K, K), 0)
    jj = lax.broadcasted_iota(_I32, (K, K), 1)
    a_hat = jnp.where(ii == jj, 1.0, a_ref[...])
    degp = jnp.sum(a_hat, axis=0)
    dinvp = lax.rsqrt(degp)
    recip = 1.0 / jnp.maximum(cnt_ref[0, :], 1.0)
    x_p = sums_ref[...] * recip[:, None]
    xw = jnp.dot(x_p, w2_ref[...], preferred_element_type=_F32)
    z = dinvp[:, None] * xw
    x_p2 = dinvp[:, None] * lax.dot_general(
        a_hat, z, (((0,), (0,)), ((), ())), preferred_element_type=_F32)
    out_ref[...] = x_p2 + b2_ref[...]


def _pgcn(A, sums, cnt, W2, b2):
    K = K_TARGET
    return pl.pallas_call(
        _pgcn_body,
        out_shape=jax.ShapeDtypeStruct((K, OUT), _F32),
    )(A, sums, cnt, W2, b2[None, :])


def _bcast_body(cid_ref, xp2_ref, skip_ref, out_ref):
    onehot = (lax.broadcasted_iota(_I32, (cid_ref.shape[0], K_TARGET), 1)
              == cid_ref[...]).astype(_F32)
    out_ref[...] = jnp.dot(onehot, xp2_ref[...],
                           preferred_element_type=_F32) + skip_ref[...]


def _bcast(cluster_id, x_p2, skip):
    B = 1000
    _i32 = lambda v: jnp.asarray(v, _I32)
    return pl.pallas_call(
        _bcast_body,
        grid=(N_NODES // B,),
        in_specs=[
            pl.BlockSpec((B, 1), lambda i: (_i32(i), _i32(0))),
            pl.BlockSpec((K_TARGET, OUT), lambda i: (_i32(0), _i32(0))),
            pl.BlockSpec((B, OUT), lambda i: (_i32(i), _i32(0))),
        ],
        out_specs=pl.BlockSpec((B, OUT), lambda i: (_i32(i), _i32(0))),
        out_shape=jax.ShapeDtypeStruct((N_NODES, OUT), _F32),
    )(cluster_id[:, None], x_p2, skip)


def kernel(x, edge_index, W1, b1, W2, b2, Wskip, bskip, Wscore):
    out_dtype = jnp.result_type(x.dtype, W1.dtype)
    x = x.astype(_F32)
    W1 = W1.astype(_F32)
    b1 = b1.astype(_F32)
    W2 = W2.astype(_F32)
    b2 = b2.astype(_F32)
    Wskip = Wskip.astype(_F32)
    bskip = bskip.astype(_F32)
    Wscore = Wscore.astype(_F32)
    src = edge_index[0].astype(_I32)
    dst = edge_index[1].astype(_I32)
    N, E, K = N_NODES, E_EDGES, K_TARGET

    deg_dst = jnp.zeros((N,), _I32).at[dst].add(1)
    deg_src = jnp.zeros((N,), _I32).at[src].add(1)
    dinv = lax.rsqrt(deg_dst.astype(_F32) + 1.0)

    y = dinv[:, None] * x
    pad = _EPAD - E
    src2d = jnp.concatenate([src, jnp.zeros((pad,), _I32)]).reshape(
        _TILES * _CPT, _CHUNK)
    dst2d = jnp.concatenate([dst, jnp.full((pad,), _NPAD - 1, _I32)]).reshape(
        _TILES * _CPT, _CHUNK)
    zrows = jnp.zeros((_RPT, _HALF), _F32)
    outA, outB = _sc_aggregate(y[:, :_HALF], y[:, _HALF:], src2d, dst2d, zrows)
    agg = jnp.concatenate([outA[:N], outB[:N]], axis=1)


    x1g, skip, raw = _dense1(agg, x, dinv, W1, b1, Wscore, Wskip, bskip)

    _, kept = lax.top_k(raw, K)
    keep_mask = jnp.zeros((N,), bool).at[kept].set(True)
    cluster_rank = jnp.cumsum(keep_mask.astype(_I32)) - 1

    maxdeg = jnp.max(jnp.where(keep_mask, deg_src, -1))
    elig = keep_mask & (deg_src == maxdeg)
    bg_node = jnp.argmax(jnp.where(elig, raw, -jnp.inf))
    best_global_cluster = cluster_rank[bg_node]

    npadding = (0, _NPAD - N)
    cid_pad, cu2d, cv2d = _sc_cluster(
        src.reshape(_TILES, E // _TILES),
        dst.reshape(_TILES, E // _TILES),
        jnp.pad(keep_mask.astype(_I32), npadding),
        jnp.pad(deg_src, npadding),
        jnp.pad(cluster_rank, npadding),
        jnp.full((16,), best_global_cluster, _I32),
    )
    cluster_id = cid_pad[:N]

    sums, cnt = _pool(cluster_id, x1g)

    A = _sc_adj(cu2d, cv2d)
    x_p2 = _pgcn(A, sums, cnt, W2, b2)

    out = _bcast(cluster_id, x_p2, skip)
    return (out.astype(out_dtype), 0.0)
